# Initial kernel scaffold; baseline (speedup 1.0000x reference)
import jax
import jax.numpy as jnp
from jax.experimental import pallas as pl


def kernel(hidden_states, topk_indices, topk_weights, W1, W2):
    raise NotImplementedError("write your pallas kernel here")


# speedup vs baseline: 1.2845x; 1.2845x over previous
import functools

import jax
import jax.numpy as jnp
from jax.experimental import pallas as pl
from jax.experimental.pallas import tpu as pltpu


def _moe_dense_body(idx_ref, wts_ref, x_ref, w1_ref, w2_ref, out_ref):
    e = pl.program_id(0)
    f = pl.program_id(1)

    @pl.when(jnp.logical_and(e == 0, f == 0))
    def _():
        out_ref[...] = jnp.zeros_like(out_ref)

    x = x_ref[...]
    w1 = w1_ref[0].astype(jnp.bfloat16)
    w2 = w2_ref[0].astype(jnp.bfloat16)
    h = jnp.dot(x, w1, preferred_element_type=jnp.float32)
    h = jax.nn.gelu(h)
    y = jnp.dot(h.astype(jnp.bfloat16), w2,
                preferred_element_type=jnp.float32)
    idx = idx_ref[...]
    wts = wts_ref[...]
    cw = jnp.sum(jnp.where(idx == e, wts, 0.0), axis=1, keepdims=True)
    out_ref[...] += y * cw


def kernel(hidden_states, topk_indices, topk_weights, W1, W2):
    n_tokens, d_model = hidden_states.shape
    n_experts, _, d_ff = W1.shape
    ff_b = min(512, d_ff)
    grid = (n_experts, d_ff // ff_b)

    x = hidden_states.astype(jnp.bfloat16)
    idx = topk_indices.astype(jnp.int32)

    out = pl.pallas_call(
        _moe_dense_body,
        grid=grid,
        in_specs=[
            pl.BlockSpec(idx.shape, lambda e, f: (0, 0)),
            pl.BlockSpec(topk_weights.shape, lambda e, f: (0, 0)),
            pl.BlockSpec((n_tokens, d_model), lambda e, f: (0, 0)),
            pl.BlockSpec((1, d_model, ff_b), lambda e, f: (e, 0, f)),
            pl.BlockSpec((1, ff_b, d_model), lambda e, f: (e, f, 0)),
        ],
        out_specs=pl.BlockSpec((n_tokens, d_model), lambda e, f: (0, 0)),
        out_shape=jax.ShapeDtypeStruct((n_tokens, d_model), jnp.float32),
        compiler_params=pltpu.CompilerParams(
            dimension_semantics=("arbitrary", "arbitrary"),
        ),
    )(idx, topk_weights, x, W1, W2)
    return out
